# baseline (speedup 1.0000x reference)
import functools

import jax
import jax.numpy as jnp
from jax import lax
from jax.experimental import pallas as pl
from jax.experimental.pallas import tpu as pltpu
from jax.experimental.pallas import tpu_sc as plsc

BATCH = 16384
NFIELDS = 26
FA = 10
FB = NFIELDS - FA
OFF_A = 15_999_616
ROWS_A = 26_000_000 - OFF_A
ROWS_B = 16_000_000
NC = 2
NS = 16
NW = NC * NS
BPW = BATCH // NW
EPA = FA * BPW
EPB = FB * BPW
JCH = BPW // 16

_mesh = plsc.VectorSubcoreMesh(core_axis_name="c", subcore_axis_name="s")


def _make_kernel1():
    @functools.partial(
        pl.kernel,
        mesh=_mesh,
        out_type=(
            jax.ShapeDtypeStruct((BATCH,), jnp.float32),
            jax.ShapeDtypeStruct((NW, EPB), jnp.int32),
        ),
        scratch_types=[
            pltpu.VMEM((NFIELDS, BPW), jnp.int32),
            pltpu.VMEM((EPA,), jnp.int32),
            pltpu.VMEM((EPB,), jnp.int32),
            pltpu.VMEM((NFIELDS * 16,), jnp.int32),
            pltpu.VMEM((EPA,), jnp.float32),
            pltpu.VMEM((16,), jnp.float32),
            pltpu.VMEM((BPW,), jnp.float32),
            pltpu.SemaphoreType.DMA,
        ],
    )
    def k1(xt_hbm, offs_hbm, tabA_hbm, bias_hbm, part_hbm, idx2_hbm,
           x_v, idxa_v, idxb_v, off_v, val_v, bias_v, out_v, sem):
        wid = lax.axis_index("s") * NC + lax.axis_index("c")
        with jax.named_scope("dma_in"):
            cp_x = pltpu.async_copy(
                xt_hbm.at[:, pl.ds(wid * BPW, BPW)], x_v, sem)
            cp_o = pltpu.async_copy(offs_hbm, off_v, sem)
            cp_b = pltpu.async_copy(bias_hbm, bias_v, sem)
            cp_x.wait()
            cp_o.wait()
            cp_b.wait()

        offv = [off_v[pl.ds(f * 16, 16)] for f in range(NFIELDS)]
        half = jnp.full((16,), OFF_A, jnp.int32)

        with jax.named_scope("add_loop"):
            def add_body(j, carry):
                c = j * 16
                for f in range(FB, NFIELDS):
                    idxa_v[pl.ds((f - FB) * BPW + c, 16)] = (
                        x_v[f, pl.ds(c, 16)] + (offv[f] - half))
                for f in range(FB):
                    idxb_v[pl.ds(f * BPW + c, 16)] = (
                        x_v[f, pl.ds(c, 16)] + offv[f])
                return carry
            lax.fori_loop(0, JCH, add_body, 0)

        with jax.named_scope("gather"):
            cp_i = pltpu.async_copy(idxb_v, idx2_hbm.at[wid], sem)
            pltpu.async_copy(tabA_hbm.at[idxa_v], val_v, sem).wait()
            cp_i.wait()

        bvec = bias_v[...]

        with jax.named_scope("reduce"):
            def red_body(j, carry):
                c = j * 16
                acc = bvec
                for f in range(FA):
                    acc = acc + val_v[pl.ds(f * BPW + c, 16)]
                out_v[pl.ds(c, 16)] = acc
                return carry
            lax.fori_loop(0, JCH, red_body, 0)

        with jax.named_scope("dma_out"):
            pltpu.sync_copy(out_v, part_hbm.at[pl.ds(wid * BPW, BPW)])

    return k1


def _make_kernel2():
    @functools.partial(
        pl.kernel,
        mesh=_mesh,
        out_type=jax.ShapeDtypeStruct((BATCH,), jnp.float32),
        scratch_types=[
            pltpu.VMEM((EPB,), jnp.int32),
            pltpu.VMEM((EPB,), jnp.float32),
            pltpu.VMEM((BPW,), jnp.float32),
            pltpu.VMEM((BPW,), jnp.float32),
            pltpu.SemaphoreType.DMA,
        ],
    )
    def k2(idx2_hbm, tabB_hbm, part_hbm, out_hbm,
           idx_v, val_v, part_v, out_v, sem):
        wid = lax.axis_index("s") * NC + lax.axis_index("c")
        with jax.named_scope("dma_in2"):
            cp_i = pltpu.async_copy(idx2_hbm.at[wid], idx_v, sem)
            cp_p = pltpu.async_copy(
                part_hbm.at[pl.ds(wid * BPW, BPW)], part_v, sem)
            cp_i.wait()

        with jax.named_scope("gather2"):
            pltpu.async_copy(tabB_hbm.at[idx_v], val_v, sem).wait()
            cp_p.wait()

        with jax.named_scope("reduce2"):
            def red_body(j, carry):
                c = j * 16
                acc = part_v[pl.ds(c, 16)]
                for f in range(FB):
                    acc = acc + val_v[pl.ds(f * BPW + c, 16)]
                out_v[pl.ds(c, 16)] = acc
                return carry
            lax.fori_loop(0, JCH, red_body, 0)

        with jax.named_scope("dma_out2"):
            pltpu.sync_copy(out_v, out_hbm.at[pl.ds(wid * BPW, BPW)])

    return k2


_gather_sum_1 = _make_kernel1()
_gather_sum_2 = _make_kernel2()


def kernel(x, offsets, table, bias):
    xt = x.T
    offs = jnp.repeat(offsets, 16)
    tabA = lax.slice(table, (OFF_A, 0), (26_000_000, 1)).reshape(ROWS_A)
    tabA, table2 = lax.optimization_barrier((tabA, table))
    tabB = lax.slice(table2, (0, 0), (ROWS_B, 1)).reshape(ROWS_B)
    bias16 = jnp.broadcast_to(bias.astype(jnp.float32), (16,))
    partial, idx2 = _gather_sum_1(xt, offs, tabA, bias16)
    out = _gather_sum_2(idx2, tabB, partial)
    return out.reshape(BATCH, 1)


# speedup vs baseline: 1.0084x; 1.0084x over previous
import functools

import jax
import jax.numpy as jnp
from jax import lax
from jax.experimental import pallas as pl
from jax.experimental.pallas import tpu as pltpu
from jax.experimental.pallas import tpu_sc as plsc

BATCH = 16384
NFIELDS = 26
FA = 16
FB = NFIELDS - FA
ROWS_A = 16_000_000
OFF_B = 15_999_616
ROWS_B = 26_000_000 - OFF_B
NC = 2
NS = 16
NW = NC * NS
BPW = BATCH // NW
EPA = FA * BPW
EPB = FB * BPW
JCH = BPW // 16

_mesh = plsc.VectorSubcoreMesh(core_axis_name="c", subcore_axis_name="s")


def _make_kernel1():
    @functools.partial(
        pl.kernel,
        mesh=_mesh,
        out_type=(
            jax.ShapeDtypeStruct((BATCH,), jnp.float32),
            jax.ShapeDtypeStruct((NW, EPB), jnp.int32),
        ),
        scratch_types=[
            pltpu.VMEM((NFIELDS, BPW), jnp.int32),
            pltpu.VMEM((EPA,), jnp.int32),
            pltpu.VMEM((EPB,), jnp.int32),
            pltpu.VMEM((NFIELDS * 16,), jnp.int32),
            pltpu.VMEM((EPA,), jnp.float32),
            pltpu.VMEM((16,), jnp.float32),
            pltpu.VMEM((BPW,), jnp.float32),
            pltpu.SemaphoreType.DMA,
        ],
    )
    def k1(xt_hbm, offs_hbm, tabA_hbm, bias_hbm, part_hbm, idx2_hbm,
           x_v, idxa_v, idxb_v, off_v, val_v, bias_v, out_v, sem):
        wid = lax.axis_index("s") * NC + lax.axis_index("c")
        with jax.named_scope("dma_in"):
            cp_x = pltpu.async_copy(
                xt_hbm.at[:, pl.ds(wid * BPW, BPW)], x_v, sem)
            cp_o = pltpu.async_copy(offs_hbm, off_v, sem)
            cp_b = pltpu.async_copy(bias_hbm, bias_v, sem)
            cp_x.wait()
            cp_o.wait()
            cp_b.wait()

        offv = [off_v[pl.ds(f * 16, 16)] for f in range(NFIELDS)]
        half = jnp.full((16,), OFF_B, jnp.int32)

        with jax.named_scope("add_loop"):
            def add_body(j, carry):
                c = j * 16
                for f in range(FA):
                    idxa_v[pl.ds(f * BPW + c, 16)] = (
                        x_v[f, pl.ds(c, 16)] + offv[f])
                for f in range(FA, NFIELDS):
                    idxb_v[pl.ds((f - FA) * BPW + c, 16)] = (
                        x_v[f, pl.ds(c, 16)] + (offv[f] - half))
                return carry
            lax.fori_loop(0, JCH, add_body, 0)

        with jax.named_scope("gather"):
            cp_i = pltpu.async_copy(idxb_v, idx2_hbm.at[wid], sem)
            pltpu.async_copy(tabA_hbm.at[idxa_v], val_v, sem).wait()
            cp_i.wait()

        bvec = bias_v[...]

        with jax.named_scope("reduce"):
            def red_body(j, carry):
                c = j * 16
                acc = bvec
                for f in range(FA):
                    acc = acc + val_v[pl.ds(f * BPW + c, 16)]
                out_v[pl.ds(c, 16)] = acc
                return carry
            lax.fori_loop(0, JCH, red_body, 0)

        with jax.named_scope("dma_out"):
            pltpu.sync_copy(out_v, part_hbm.at[pl.ds(wid * BPW, BPW)])

    return k1


def _make_kernel2():
    @functools.partial(
        pl.kernel,
        mesh=_mesh,
        out_type=jax.ShapeDtypeStruct((BATCH,), jnp.float32),
        scratch_types=[
            pltpu.VMEM((EPB,), jnp.int32),
            pltpu.VMEM((EPB,), jnp.float32),
            pltpu.VMEM((BPW,), jnp.float32),
            pltpu.VMEM((BPW,), jnp.float32),
            pltpu.SemaphoreType.DMA,
        ],
    )
    def k2(idx2_hbm, tabB_hbm, part_hbm, out_hbm,
           idx_v, val_v, part_v, out_v, sem):
        wid = lax.axis_index("s") * NC + lax.axis_index("c")
        with jax.named_scope("dma_in2"):
            cp_i = pltpu.async_copy(idx2_hbm.at[wid], idx_v, sem)
            cp_p = pltpu.async_copy(
                part_hbm.at[pl.ds(wid * BPW, BPW)], part_v, sem)
            cp_i.wait()

        with jax.named_scope("gather2"):
            pltpu.async_copy(tabB_hbm.at[idx_v], val_v, sem).wait()
            cp_p.wait()

        with jax.named_scope("reduce2"):
            def red_body(j, carry):
                c = j * 16
                acc = part_v[pl.ds(c, 16)]
                for f in range(FB):
                    acc = acc + val_v[pl.ds(f * BPW + c, 16)]
                out_v[pl.ds(c, 16)] = acc
                return carry
            lax.fori_loop(0, JCH, red_body, 0)

        with jax.named_scope("dma_out2"):
            pltpu.sync_copy(out_v, out_hbm.at[pl.ds(wid * BPW, BPW)])

    return k2


_gather_sum_1 = _make_kernel1()
_gather_sum_2 = _make_kernel2()


def kernel(x, offsets, table, bias):
    xt = x.T
    offs = jnp.repeat(offsets, 16)
    tabA = lax.slice(table, (0, 0), (ROWS_A, 1)).reshape(ROWS_A)
    tabA, table2 = lax.optimization_barrier((tabA, table))
    tabB = lax.slice(table2, (OFF_B, 0), (26_000_000, 1)).reshape(ROWS_B)
    bias16 = jnp.broadcast_to(bias.astype(jnp.float32), (16,))
    partial, idx2 = _gather_sum_1(xt, offs, tabA, bias16)
    out = _gather_sum_2(idx2, tabB, partial)
    return out.reshape(BATCH, 1)
